# Initial kernel scaffold; baseline (speedup 1.0000x reference)
import jax
import jax.numpy as jnp
from jax.experimental import pallas as pl


def kernel(x, batch, y, W1, b1, W2, b2):
    raise NotImplementedError("write your pallas kernel here")


# speedup vs baseline: 4.0953x; 4.0953x over previous
import functools

import jax
import jax.numpy as jnp
from jax import lax
from jax.experimental import pallas as pl
from jax.experimental.pallas import tpu as pltpu
from jax.experimental.pallas import tpu_sc as plsc

_G = 256
_N = 100000
_D = 128
_NC = 2
_NS = 16
_NW = _NC * _NS
_BLK = 128
_NBLKS_TOTAL = (_N + _BLK - 1) // _BLK
_BASE_BLKS = _NBLKS_TOTAL // _NW
_EXTRA = _NBLKS_TOTAL - _BASE_BLKS * _NW
_MAX_BLKS = _BASE_BLKS + 1


def _sc_segment_sum(x, batch):
    mesh = plsc.VectorSubcoreMesh(core_axis_name="c", subcore_axis_name="s")

    @functools.partial(
        pl.kernel,
        out_type=jax.ShapeDtypeStruct((_NC, _G, _D), jnp.float32),
        mesh=mesh,
        scratch_types=[
            pltpu.VMEM((_BLK, _D), jnp.float32),
            pltpu.VMEM((_BLK,), jnp.int32),
            pltpu.VMEM_SHARED((_G + 8, _D), jnp.float32),
        ],
    )
    def seg_sum(x_hbm, b_hbm, out_hbm, xbuf, ibuf, acc):
        cid = lax.axis_index("c")
        sid = lax.axis_index("s")
        wid = sid * _NC + cid

        @pl.when(sid == 0)
        def _zero():
            zeros = jnp.zeros((16,), jnp.float32)

            def zrow(j, _):
                for i in range(_D // 16):
                    xbuf[j, pl.ds(i * 16, 16)] = zeros
                return 0

            lax.fori_loop(0, _BLK, zrow, 0)
            pltpu.sync_copy(xbuf, acc.at[pl.ds(0, _BLK)])
            pltpu.sync_copy(xbuf, acc.at[pl.ds(_BLK, _BLK)])

        plsc.subcore_barrier()

        base = _BASE_BLKS * wid + jnp.minimum(wid, _EXTRA)
        nblk = jnp.where(wid < _EXTRA, _MAX_BLKS, _BASE_BLKS)

        def body(b, _):
            @pl.when(b < nblk)
            def _do():
                gstart = (base + b) * _BLK
                xstart = jnp.minimum(gstart, _N - _BLK)
                pltpu.sync_copy(x_hbm.at[pl.ds(xstart, _BLK)], xbuf)
                pltpu.sync_copy(b_hbm.at[pl.ds(xstart, _BLK)], ibuf)
                for i in range(_BLK // 16):
                    r = xstart + i * 16 + lax.iota(jnp.int32, 16)
                    v = ibuf[pl.ds(i * 16, 16)]
                    ibuf[pl.ds(i * 16, 16)] = jnp.where(r >= gstart, v, _G)
                pltpu.sync_copy(xbuf, acc.at[ibuf], add=True)

            return 0

        lax.fori_loop(0, _MAX_BLKS, body, 0)

        plsc.subcore_barrier()

        @pl.when(sid == 0)
        def _readout():
            pltpu.sync_copy(acc.at[pl.ds(0, _G)], out_hbm.at[cid])

    return seg_sum(x, batch)


def _tc_mlp(partials, W1, b1, W2p, b2):
    def mlp(p_ref, w1_ref, b1_ref, w2_ref, b2_ref, o_ref):
        emb = p_ref[0] + p_ref[1]
        h = jnp.maximum(
            jnp.dot(emb, w1_ref[...], preferred_element_type=jnp.float32)
            + b1_ref[...], 0.0)
        o_ref[...] = (
            jnp.dot(h, w2_ref[...], preferred_element_type=jnp.float32)
            + b2_ref[...])

    return pl.pallas_call(
        mlp,
        out_shape=jax.ShapeDtypeStruct((_G, _D), jnp.float32),
    )(partials, W1, b1, W2p, b2)


def kernel(x, batch, y, W1, b1, W2, b2):
    partials = _sc_segment_sum(x, batch.astype(jnp.int32))
    W2p = jnp.pad(W2, ((0, 0), (0, _D - W2.shape[1])))
    b2p = jnp.pad(b2, (0, _D - b2.shape[0]))
    out = _tc_mlp(partials, W1, b1.reshape(1, _D), W2p, b2p.reshape(1, _D))
    pred = out[:, : W2.shape[1]]
    return (pred, y)


# speedup vs baseline: 6.1296x; 1.4967x over previous
import functools

import jax
import jax.numpy as jnp
from jax import lax
from jax.experimental import pallas as pl
from jax.experimental.pallas import tpu as pltpu
from jax.experimental.pallas import tpu_sc as plsc

_G = 256
_N = 100000
_D = 128
_NC = 2
_NS = 16
_NW = _NC * _NS
_BLK = 128
_NBLKS_TOTAL = (_N + _BLK - 1) // _BLK
_BASE_BLKS = _NBLKS_TOTAL // _NW
_EXTRA = _NBLKS_TOTAL - _BASE_BLKS * _NW
_MAX_BLKS = _BASE_BLKS + 1
_CHUNK = _MAX_BLKS * _BLK
_ZROWS = _G // _NS


def _sc_segment_sum(x, batch):
    mesh = plsc.VectorSubcoreMesh(core_axis_name="c", subcore_axis_name="s")

    @functools.partial(
        pl.kernel,
        out_type=jax.ShapeDtypeStruct((_NC, _G, _D), jnp.float32),
        mesh=mesh,
        scratch_types=[
            pltpu.VMEM((2, _BLK, _D), jnp.float32),
            pltpu.VMEM((_CHUNK,), jnp.int32),
            pltpu.VMEM((2, _BLK), jnp.int32),
            pltpu.VMEM((_ZROWS, _D), jnp.float32),
            pltpu.VMEM_SHARED((_G + 8, _D), jnp.float32),
            pltpu.SemaphoreType.DMA((2,)),
        ],
    )
    def seg_sum(x_hbm, b_hbm, out_hbm, xbuf, ibuf, idx2, zbuf, acc, sems):
        cid = lax.axis_index("c")
        sid = lax.axis_index("s")
        wid = sid * _NC + cid

        zeros = jnp.zeros((16,), jnp.float32)

        def zrow(j, _):
            for i in range(_D // 16):
                zbuf[j, pl.ds(i * 16, 16)] = zeros
            return 0

        lax.fori_loop(0, _ZROWS, zrow, 0)
        pltpu.sync_copy(zbuf, acc.at[pl.ds(sid * _ZROWS, _ZROWS)])

        base = _BASE_BLKS * wid + jnp.minimum(wid, _EXTRA)
        nblk = jnp.where(wid < _EXTRA, _MAX_BLKS, _BASE_BLKS)

        cstart = jnp.minimum(base * _BLK, _N - _CHUNK)
        pltpu.sync_copy(b_hbm.at[pl.ds(cstart, _CHUNK)], ibuf)

        plsc.subcore_barrier()

        def xsrc(b):
            gstart = (base + b) * _BLK
            return x_hbm.at[pl.ds(jnp.minimum(gstart, _N - _BLK), _BLK)]

        def start(b, slot):
            pltpu.async_copy(xsrc(b), xbuf.at[slot], sems.at[slot])

        start(0, 0)

        def body(b, _):
            slot = lax.rem(b, 2)

            @pl.when(b + 1 < nblk)
            def _prefetch():
                start(b + 1, lax.rem(b + 1, 2))

            @pl.when(b < nblk)
            def _process():
                gstart = (base + b) * _BLK
                xstart = jnp.minimum(gstart, _N - _BLK)
                pltpu.make_async_copy(xsrc(b), xbuf.at[slot], sems.at[slot]).wait()
                for i in range(_BLK // 16):
                    r = xstart + i * 16 + lax.iota(jnp.int32, 16)
                    v = ibuf[pl.ds(xstart - cstart + i * 16, 16)]
                    idx2[slot, pl.ds(i * 16, 16)] = jnp.where(r >= gstart, v, _G)
                pltpu.sync_copy(xbuf.at[slot], acc.at[idx2.at[slot]], add=True)

            return 0

        lax.fori_loop(0, _MAX_BLKS, body, 0)

        plsc.subcore_barrier()

        @pl.when(sid == 0)
        def _readout():
            pltpu.sync_copy(acc.at[pl.ds(0, _G)], out_hbm.at[cid])

    return seg_sum(x, batch)


def _tc_mlp(partials, W1, b1, W2p, b2):
    def mlp(p_ref, w1_ref, b1_ref, w2_ref, b2_ref, o_ref):
        emb = p_ref[0] + p_ref[1]
        h = jnp.maximum(
            jnp.dot(emb, w1_ref[...], preferred_element_type=jnp.float32)
            + b1_ref[...], 0.0)
        o_ref[...] = (
            jnp.dot(h, w2_ref[...], preferred_element_type=jnp.float32)
            + b2_ref[...])

    return pl.pallas_call(
        mlp,
        out_shape=jax.ShapeDtypeStruct((_G, _D), jnp.float32),
    )(partials, W1, b1, W2p, b2)


def kernel(x, batch, y, W1, b1, W2, b2):
    partials = _sc_segment_sum(x, batch.astype(jnp.int32))
    W2p = jnp.pad(W2, ((0, 0), (0, _D - W2.shape[1])))
    b2p = jnp.pad(b2, (0, _D - b2.shape[0]))
    out = _tc_mlp(partials, W1, b1.reshape(1, _D), W2p, b2p.reshape(1, _D))
    pred = out[:, : W2.shape[1]]
    return (pred, y)
